# baseline (speedup 1.0000x reference)
import functools
import jax
import jax.numpy as jnp
from jax import lax
from jax.experimental import pallas as pl
from jax.experimental.pallas import tpu as pltpu
from jax.experimental.pallas import tpu_sc as plsc

_B, _S, _D = 4, 8192, 1024
_NW = 32
_SC_ROWS = _S
_SC_BASE = 3 * _S
_PER_W = _SC_ROWS // _NW
_T = 16
_STEPS = _PER_W // _T

_SEQ_BLK = 2048


def _tc_body(x_ref, emb_ref, o_ref):
    o_ref[...] = x_ref[...] + emb_ref[...]


def _tc_part(x, emb):
    return pl.pallas_call(
        _tc_body,
        grid=(_S // _SEQ_BLK, 3),
        in_specs=[
            pl.BlockSpec((1, _SEQ_BLK, _D), lambda i, j: (j, i, 0)),
            pl.BlockSpec((_SEQ_BLK, _D), lambda i, j: (i, 0)),
        ],
        out_specs=pl.BlockSpec((1, _SEQ_BLK, _D), lambda i, j: (j, i, 0)),
        out_shape=jax.ShapeDtypeStruct((3, _S, _D), jnp.float32),
    )(x, emb)


@functools.partial(
    pl.kernel,
    mesh=plsc.VectorSubcoreMesh(core_axis_name="c", subcore_axis_name="s"),
    out_type=jax.ShapeDtypeStruct((_SC_ROWS, _D), jnp.float32),
    scratch_types=[
        pltpu.VMEM((2, _T, _D), jnp.float32),
        pltpu.VMEM((2, _T, _D), jnp.float32),
        pltpu.VMEM((2, _T, _D), jnp.float32),
        pltpu.SemaphoreType.DMA,
        pltpu.SemaphoreType.DMA,
        pltpu.SemaphoreType.DMA,
        pltpu.SemaphoreType.DMA,
        pltpu.SemaphoreType.DMA,
        pltpu.SemaphoreType.DMA,
    ],
)
def _sc_part(x_hbm, emb_hbm, out_hbm, xbuf, ebuf, obuf,
             sx0, sx1, se0, se1, so0, so1):
    sx, se, so = [sx0, sx1], [se0, se1], [so0, so1]
    wid = lax.axis_index("s") * 2 + lax.axis_index("c")
    base = wid * _PER_W

    def x_slice(t):
        return x_hbm.at[pl.ds(_SC_BASE + base + t * _T, _T), :]

    def e_slice(t):
        return emb_hbm.at[pl.ds(base + t * _T, _T), :]

    def o_slice(t):
        return out_hbm.at[pl.ds(base + t * _T, _T), :]

    for b in range(2):
        pltpu.async_copy(x_slice(b), xbuf.at[b], sx[b])
        pltpu.async_copy(e_slice(b), ebuf.at[b], se[b])

    def outer(g, _):
        for b in range(2):
            t = 2 * g + b
            pltpu.make_async_copy(x_slice(t), xbuf.at[b], sx[b]).wait()
            pltpu.make_async_copy(e_slice(t), ebuf.at[b], se[b]).wait()

            @pl.when(g > 0)
            def _wait_store():
                pltpu.make_async_copy(obuf.at[b], o_slice(t - 2), so[b]).wait()

            def add_row(r, _):
                for u in range(_D // 16):
                    sl = pl.ds(u * 16, 16)
                    obuf[b, r, sl] = xbuf[b, r, sl] + ebuf[b, r, sl]
                return 0

            lax.fori_loop(0, _T, add_row, 0)

            pltpu.async_copy(obuf.at[b], o_slice(t), so[b])

            @pl.when(g < _STEPS // 2 - 1)
            def _next_loads():
                pltpu.async_copy(x_slice(t + 2), xbuf.at[b], sx[b])
                pltpu.async_copy(e_slice(t + 2), ebuf.at[b], se[b])

        return 0

    lax.fori_loop(0, _STEPS // 2, outer, 0)

    for b in range(2):
        pltpu.make_async_copy(obuf.at[b], o_slice(_STEPS - 2 + b), so[b]).wait()


def kernel(x, emb):
    B, S, D = x.shape
    out_tc = _tc_part(x, emb)
    out_sc = _sc_part(x.reshape(B * S, D), emb)
    return jnp.concatenate([out_tc, out_sc.reshape(1, S, D)], axis=0)


# speedup vs baseline: 2.2063x; 2.2063x over previous
import jax
import jax.numpy as jnp
from jax.experimental import pallas as pl

_SEQ_BLK = 1024


def _add_kernel(x_ref, emb_ref, o_ref):
    o_ref[...] = x_ref[...] + emb_ref[...]


def kernel(x, emb):
    B, S, D = x.shape
    grid = (S // _SEQ_BLK, B // 2)
    return pl.pallas_call(
        _add_kernel,
        grid=grid,
        in_specs=[
            pl.BlockSpec((2, _SEQ_BLK, D), lambda i, j: (j, i, 0)),
            pl.BlockSpec((_SEQ_BLK, D), lambda i, j: (i, 0)),
        ],
        out_specs=pl.BlockSpec((2, _SEQ_BLK, D), lambda i, j: (j, i, 0)),
        out_shape=jax.ShapeDtypeStruct((B, S, D), x.dtype),
    )(x, emb)
